# baseline (speedup 1.0000x reference)
import functools

import jax
import jax.numpy as jnp
from jax import lax
from jax.experimental import pallas as pl
from jax.experimental.pallas import tpu as pltpu
from jax.experimental.pallas import tpu_sc as plsc

IC = 128
NGROUP = 32
CPG = IC // NGROUP
EPSV = 1e-5
NSEG = 16

BLK = 80
NW = 32


def _sc_stats_body(nblk, data_h, bid_h, s1_o, s2_o,
                   x_v, x2_v, bid_v, z_v, s1_sh, s2_sh):
    c = lax.axis_index("c")
    s = lax.axis_index("s")
    w = s * 2 + c

    @pl.when(s == 0)
    def _():
        zv = jnp.zeros((16,), jnp.float32)
        for r in range(NSEG):
            for q in range(IC // 16):
                z_v[r, pl.ds(q * 16, 16)] = zv
        pltpu.sync_copy(z_v, s1_sh)
        pltpu.sync_copy(z_v, s2_sh)

    plsc.subcore_barrier()

    def iter_body(it, carry):
        blk = it * NW + w

        @pl.when(blk < nblk)
        def _():
            base = blk * BLK
            pltpu.sync_copy(bid_h.at[pl.ds(base, BLK)], bid_v)
            pltpu.sync_copy(data_h.at[pl.ds(base, BLK)], x_v)
            pltpu.sync_copy(x_v, s1_sh.at[bid_v], add=True)

            def row_body(r, rc):
                for q in range(IC // 16):
                    v = x_v[r, pl.ds(q * 16, 16)]
                    x2_v[r, pl.ds(q * 16, 16)] = v * v
                return rc

            lax.fori_loop(0, BLK, row_body, 0)
            pltpu.sync_copy(x2_v, s2_sh.at[bid_v], add=True)

        return carry

    lax.fori_loop(0, (nblk + NW - 1) // NW, iter_body, 0)
    plsc.subcore_barrier()

    @pl.when(s == 0)
    def _():
        pltpu.sync_copy(s1_sh, s1_o.at[c])
        pltpu.sync_copy(s2_sh, s2_o.at[c])


def _sc_stats(data, bid):
    n = data.shape[0]
    nblk = n // BLK
    assert nblk * BLK == n
    mesh = plsc.VectorSubcoreMesh(core_axis_name="c", subcore_axis_name="s")
    f = pl.kernel(
        functools.partial(_sc_stats_body, nblk),
        out_type=[
            jax.ShapeDtypeStruct((2, NSEG, IC), jnp.float32),
            jax.ShapeDtypeStruct((2, NSEG, IC), jnp.float32),
        ],
        mesh=mesh,
        scratch_types=[
            pltpu.VMEM((BLK, IC), jnp.float32),
            pltpu.VMEM((BLK, IC), jnp.float32),
            pltpu.VMEM((BLK,), jnp.int32),
            pltpu.VMEM((NSEG, IC), jnp.float32),
            pltpu.VMEM_SHARED((NSEG, IC), jnp.float32),
            pltpu.VMEM_SHARED((NSEG, IC), jnp.float32),
        ],
    )
    return f(data, bid)


def _gpool(t):
    s = t
    for k in range(1, CPG):
        s = s + pltpu.roll(t, IC - k, 1)
    lane = lax.broadcasted_iota(jnp.int32, t.shape, 1)
    lead = jnp.where(lane % CPG == 0, s, 0.0)
    s = lead
    for k in range(1, CPG):
        s = s + pltpu.roll(lead, k, 1)
    return s


def _seg_mask(bid_row, rows):
    seg = lax.broadcasted_iota(jnp.int32, (NSEG, rows), 0)
    return bid_row[None, :] == seg


def _apply_body(rows, s1p_ref, s2p_ref, cnt_ref, x_ref, bid_ref,
                w_ref, b_ref, o_ref, tab):
    j = pl.program_id(0)

    @pl.when(j == 0)
    def _():
        s1c = s1p_ref[0] + s1p_ref[1]
        s2c = s2p_ref[0] + s2p_ref[1]
        cc = cnt_ref[...]
        ic = 1.0 / (cc * CPG + EPSV)
        a1 = _gpool(s1c)
        a2 = _gpool(s2c)
        mg = a1 * ic
        var = ic * (a2 - 2.0 * mg * a1 + cc * CPG * mg * mg)
        istd = lax.rsqrt(var + EPSV)
        scale = istd * w_ref[...]
        shift = b_ref[...] - mg * scale
        tab[...] = jnp.concatenate([scale, shift],
                                   axis=1).astype(jnp.bfloat16)

    x = x_ref[...]
    oht = _seg_mask(bid_ref[0, 0, :], rows).astype(jnp.bfloat16)
    rsh = lax.dot_general(oht, tab[...], (((0,), (0,)), ((), ())),
                          preferred_element_type=jnp.float32)
    o_ref[...] = x * rsh[:, :IC] + rsh[:, IC:]


def kernel(data, batch_id, batch_size, weights, bias):
    n, c = data.shape
    bid = batch_id.astype(jnp.int32)
    s1p, s2p = _sc_stats(data, bid)
    seg_start = jnp.searchsorted(bid, jnp.arange(NSEG + 1), side="left")
    cnt = (seg_start[1:] - seg_start[:-1]).astype(jnp.float32).reshape(NSEG, 1)

    rows = 2000
    nblocks = n // rows
    bid3 = bid.reshape(nblocks, 1, rows)
    out = pl.pallas_call(
        functools.partial(_apply_body, rows),
        grid=(nblocks,),
        in_specs=[
            pl.BlockSpec((2, NSEG, c), lambda j: (0, 0, 0)),
            pl.BlockSpec((2, NSEG, c), lambda j: (0, 0, 0)),
            pl.BlockSpec((NSEG, 1), lambda j: (0, 0)),
            pl.BlockSpec((rows, c), lambda j: (j, 0)),
            pl.BlockSpec((1, 1, rows), lambda j: (j, 0, 0)),
            pl.BlockSpec((1, c), lambda j: (0, 0)),
            pl.BlockSpec((1, c), lambda j: (0, 0)),
        ],
        out_specs=pl.BlockSpec((rows, c), lambda j: (j, 0)),
        out_shape=jax.ShapeDtypeStruct((n, c), jnp.float32),
        scratch_shapes=[pltpu.VMEM((NSEG, 2 * c), jnp.bfloat16)],
        compiler_params=pltpu.CompilerParams(
            dimension_semantics=("arbitrary",)),
    )(s1p, s2p, cnt, data, bid3, weights, bias)
    return out


# speedup vs baseline: 1.8647x; 1.8647x over previous
import functools

import jax
import jax.numpy as jnp
from jax import lax
from jax.experimental import pallas as pl
from jax.experimental.pallas import tpu as pltpu

IC = 128
NGROUP = 32
CPG = IC // NGROUP
EPSV = 1e-5
NSEG = 16


def _seg_mask(bid_row, rows):
    seg = lax.broadcasted_iota(jnp.int32, (NSEG, rows), 0)
    return bid_row[None, :] == seg


def _gpool(t):
    s = t
    for k in range(1, CPG):
        s = s + pltpu.roll(t, IC - k, 1)
    lane = lax.broadcasted_iota(jnp.int32, t.shape, 1)
    lead = jnp.where(lane % CPG == 0, s, 0.0)
    s = lead
    for k in range(1, CPG):
        s = s + pltpu.roll(lead, k, 1)
    return s


def _stats_body(nblocks, rows, x_ref, bid_ref, w_ref, b_ref, tab_ref,
                s1, s2, cnt):
    j = pl.program_id(0)

    @pl.when(j == 0)
    def _():
        s1[...] = jnp.zeros_like(s1)
        s2[...] = jnp.zeros_like(s2)
        cnt[...] = jnp.zeros_like(cnt)

    x = x_ref[...]
    mask = _seg_mask(bid_ref[0, 0, :], rows)
    oht = mask.astype(jnp.bfloat16)
    xb = x.astype(jnp.bfloat16)
    s1[...] += lax.dot_general(oht, xb, (((1,), (0,)), ((), ())),
                               preferred_element_type=jnp.float32)
    s2[...] += lax.dot_general(oht, xb * xb, (((1,), (0,)), ((), ())),
                               preferred_element_type=jnp.float32)
    cnt[...] += jnp.sum(mask.astype(jnp.float32), axis=1)[:, None]

    @pl.when(j == nblocks - 1)
    def _():
        ic = 1.0 / (cnt[...] * CPG + EPSV)
        a1 = _gpool(s1[...])
        a2 = _gpool(s2[...])
        mg = a1 * ic
        var = ic * (a2 - 2.0 * mg * a1 + cnt[...] * CPG * mg * mg)
        istd = lax.rsqrt(var + EPSV)
        scale = istd * w_ref[...]
        shift = b_ref[...] - mg * scale
        tab_ref[...] = jnp.concatenate([scale, shift],
                                       axis=1).astype(jnp.bfloat16)


def _apply_body(rows, x_ref, bid_ref, tab_ref, o_ref):
    x = x_ref[...]
    oht = _seg_mask(bid_ref[0, 0, :], rows).astype(jnp.bfloat16)
    rsh = lax.dot_general(oht, tab_ref[...], (((0,), (0,)), ((), ())),
                          preferred_element_type=jnp.float32)
    o_ref[...] = x * rsh[:, :IC] + rsh[:, IC:]


def kernel(data, batch_id, batch_size, weights, bias):
    n, c = data.shape
    rows = 2000
    nblocks = n // rows
    assert nblocks * rows == n
    bid3 = batch_id.astype(jnp.int32).reshape(nblocks, 1, rows)

    tab = pl.pallas_call(
        functools.partial(_stats_body, nblocks, rows),
        grid=(nblocks,),
        in_specs=[
            pl.BlockSpec((rows, c), lambda j: (j, 0)),
            pl.BlockSpec((1, 1, rows), lambda j: (j, 0, 0)),
            pl.BlockSpec((1, c), lambda j: (0, 0)),
            pl.BlockSpec((1, c), lambda j: (0, 0)),
        ],
        out_specs=pl.BlockSpec((NSEG, 2 * c), lambda j: (0, 0)),
        out_shape=jax.ShapeDtypeStruct((NSEG, 2 * c), jnp.bfloat16),
        scratch_shapes=[
            pltpu.VMEM((NSEG, c), jnp.float32),
            pltpu.VMEM((NSEG, c), jnp.float32),
            pltpu.VMEM((NSEG, c), jnp.float32),
        ],
        compiler_params=pltpu.CompilerParams(
            dimension_semantics=("arbitrary",)),
    )(data, bid3, weights, bias)

    out = pl.pallas_call(
        functools.partial(_apply_body, rows),
        grid=(nblocks,),
        in_specs=[
            pl.BlockSpec((rows, c), lambda j: (j, 0)),
            pl.BlockSpec((1, 1, rows), lambda j: (j, 0, 0)),
            pl.BlockSpec((NSEG, 2 * c), lambda j: (0, 0)),
        ],
        out_specs=pl.BlockSpec((rows, c), lambda j: (j, 0)),
        out_shape=jax.ShapeDtypeStruct((n, c), jnp.float32),
        compiler_params=pltpu.CompilerParams(
            dimension_semantics=("arbitrary",)),
    )(data, bid3, tab)
    return out


# speedup vs baseline: 2.2374x; 1.1999x over previous
import functools

import jax
import jax.numpy as jnp
from jax import lax
from jax.experimental import pallas as pl
from jax.experimental.pallas import tpu as pltpu

IC = 128
NGROUP = 32
CPG = IC // NGROUP
EPSV = 1e-5
NSEG = 16


def _seg_mask(bid_row, rows):
    seg = lax.broadcasted_iota(jnp.int32, (NSEG, rows), 0)
    return bid_row[None, :] == seg


def _body(nblocks, rows, x_ref, bid_ref, w_ref, b_ref, o_ref,
          xs, s1, s2, cnt, tab):
    p = pl.program_id(0)
    j = pl.program_id(1)

    @pl.when((p == 0) & (j == 0))
    def _():
        s1[...] = jnp.zeros_like(s1)
        s2[...] = jnp.zeros_like(s2)
        cnt[...] = jnp.zeros_like(cnt)

    @pl.when(p == 0)
    def _():
        x = x_ref[...]
        xs[pl.ds(j * rows, rows), :] = x
        mask = _seg_mask(bid_ref[0, 0, :], rows)
        oht = mask.astype(jnp.bfloat16)
        xb = x.astype(jnp.bfloat16)
        s1[...] += lax.dot_general(oht, xb, (((1,), (0,)), ((), ())),
                                   preferred_element_type=jnp.float32)
        s2[...] += lax.dot_general(oht, xb * xb, (((1,), (0,)), ((), ())),
                                   preferred_element_type=jnp.float32)
        cnt[...] += jnp.sum(mask.astype(jnp.float32), axis=1)[:, None]

        @pl.when(j == nblocks - 1)
        def _():
            ic = 1.0 / (cnt[...] * CPG + EPSV)
            ci = lax.broadcasted_iota(jnp.int32, (IC, IC), 0) // CPG
            cj = lax.broadcasted_iota(jnp.int32, (IC, IC), 1) // CPG
            ggt = (ci == cj).astype(jnp.float32)
            a1 = lax.dot_general(s1[...], ggt, (((1,), (0,)), ((), ())),
                                 preferred_element_type=jnp.float32)
            a2 = lax.dot_general(s2[...], ggt, (((1,), (0,)), ((), ())),
                                 preferred_element_type=jnp.float32)
            mg = a1 * ic
            var = ic * (a2 - 2.0 * mg * a1 + cnt[...] * CPG * mg * mg)
            istd = lax.rsqrt(var + EPSV)
            w = w_ref[...]
            scale = istd * w
            shift = b_ref[...] - mg * scale
            tab[...] = jnp.concatenate([scale, shift],
                                       axis=1).astype(jnp.bfloat16)

    @pl.when(p == 1)
    def _():
        x = xs[pl.ds(j * rows, rows), :]
        oht = _seg_mask(bid_ref[0, 0, :], rows).astype(jnp.bfloat16)
        rsh = lax.dot_general(oht, tab[...], (((0,), (0,)), ((), ())),
                              preferred_element_type=jnp.float32)
        o_ref[...] = x * rsh[:, :IC] + rsh[:, IC:]


def kernel(data, batch_id, batch_size, weights, bias):
    n, c = data.shape
    rows = 2000
    nblocks = n // rows
    assert nblocks * rows == n
    bid3 = batch_id.astype(jnp.int32).reshape(nblocks, 1, rows)

    out = pl.pallas_call(
        functools.partial(_body, nblocks, rows),
        grid=(2, nblocks),
        in_specs=[
            pl.BlockSpec((rows, c), lambda p, j: (jnp.where(p == 0, j, 0), 0)),
            pl.BlockSpec((1, 1, rows), lambda p, j: (j, 0, 0)),
            pl.BlockSpec((1, c), lambda p, j: (0, 0)),
            pl.BlockSpec((1, c), lambda p, j: (0, 0)),
        ],
        out_specs=pl.BlockSpec((rows, c),
                               lambda p, j: (jnp.where(p == 0, 0, j), 0)),
        out_shape=jax.ShapeDtypeStruct((n, c), jnp.float32),
        scratch_shapes=[
            pltpu.VMEM((n, c), jnp.float32),
            pltpu.VMEM((NSEG, c), jnp.float32),
            pltpu.VMEM((NSEG, c), jnp.float32),
            pltpu.VMEM((NSEG, c), jnp.float32),
            pltpu.VMEM((NSEG, 2 * c), jnp.bfloat16),
        ],
        compiler_params=pltpu.CompilerParams(
            dimension_semantics=("arbitrary", "arbitrary")),
    )(data, bid3, weights, bias)
    return out
